# baseline (speedup 1.0000x reference)
import functools

import jax
import jax.numpy as jnp
from jax import lax
from jax.experimental import pallas as pl
from jax.experimental.pallas import tpu as pltpu
from jax.experimental.pallas import tpu_sc as plsc

N = 10000
E = 160000
B = 4
HID = 64
LATENT = 64
F0 = 8
F1 = 16
FOUT = 3
CH = 16

NSUB = 16
NCORE = 2
NPN = 640
NPAD = NSUB * NPN
EPT = 10240
ECH = 128
NCH = EPT // ECH
EPAD = NSUB * EPT

_SC_PARAMS = pltpu.CompilerParams(use_tc_tiling_on_sc=False)


def _elu(x):
    return jnp.where(x > 0, x, jnp.exp(x) - 1.0)


def _scdeg_body(est, dr_out, dc_out, idxv, dz, ones, dg):
    core = lax.axis_index("c")
    sub = lax.axis_index("s")
    base = sub * NPN

    pltpu.sync_copy(est.at[core, sub], idxv)

    zf = jnp.zeros((16,), jnp.float32)
    one = jnp.ones((16,), jnp.float32)

    @pl.loop(0, NPN // 16)
    def _(i):
        dz[pl.ds(16 * i, 16)] = zf

    for q in range(ECH // 16):
        ones[pl.ds(16 * q, 16)] = one

    pltpu.sync_copy(dz, dg.at[pl.ds(base, NPN)])
    plsc.subcore_barrier()

    @pl.loop(0, NCH)
    def _(j):
        pltpu.sync_copy(ones, dg.at[idxv.at[j]], add=True)

    plsc.subcore_barrier()

    @pl.when(core == 0)
    def _():
        pltpu.sync_copy(dg.at[pl.ds(base, NPN)], dr_out.at[pl.ds(base, NPN)])

    @pl.when(core == 1)
    def _():
        pltpu.sync_copy(dg.at[pl.ds(base, NPN)], dc_out.at[pl.ds(base, NPN)])


def _scdeg(est):
    mesh = plsc.VectorSubcoreMesh(core_axis_name="c", subcore_axis_name="s")
    vec = jax.ShapeDtypeStruct((NPAD,), jnp.float32)
    scratch = [
        pltpu.VMEM((NCH, ECH), jnp.int32),
        pltpu.VMEM((NPN,), jnp.float32),
        pltpu.VMEM((ECH,), jnp.float32),
        pltpu.VMEM_SHARED((NPAD,), jnp.float32),
    ]
    fn = pl.kernel(_scdeg_body, out_type=[vec, vec], mesh=mesh,
                   scratch_types=scratch, compiler_params=_SC_PARAMS)
    return fn(est)


def _tc1_body(mu, w0, b0, w1, b1, w2, b2, w3, b3, wfc, bfc, out):
    h = _elu(mu[...] @ w0[...] + b0[...])
    h = _elu(h @ w1[...] + b1[...])
    h = _elu(h @ w2[...] + b2[...])
    z = h @ w3[...] + b3[...]
    x = z @ wfc[...] + bfc[...]
    xr = x.reshape(B, x.shape[1] // F0, F0)
    out[0] = jnp.concatenate([xr[0], xr[1]], axis=-1)
    out[1] = jnp.concatenate([xr[2], xr[3]], axis=-1)


def _tc1(mu, w0, b0, w1, b1, w2, b2, w3, b3, wfc, bfc):
    grid = 4
    npb = NPAD // grid
    cols = npb * F0
    full = lambda *shape: pl.BlockSpec(shape, lambda i: (0,) * len(shape))
    return pl.pallas_call(
        _tc1_body,
        grid=(grid,),
        in_specs=[
            full(B, 8), full(8, HID), full(HID,), full(HID, HID), full(HID,),
            full(HID, HID), full(HID,), full(HID, LATENT), full(LATENT,),
            pl.BlockSpec((LATENT, cols), lambda i: (0, i)),
            pl.BlockSpec((cols,), lambda i: (i,)),
        ],
        out_specs=pl.BlockSpec((NCORE, npb, CH), lambda i: (0, i, 0)),
        out_shape=jax.ShapeDtypeStruct((NCORE, NPAD, CH), jnp.float32),
    )(mu, w0, b0, w1, b1, w2, b2, w3, b3, wfc, bfc)


def _tcab_body(dr, dc, a_out, b_out):
    a_out[...] = -lax.rsqrt(jnp.maximum(dr[...], 1.0))
    b_out[...] = lax.rsqrt(jnp.maximum(dc[...], 1.0))


def _tcab(dr, dc):
    vec = jax.ShapeDtypeStruct((NPAD,), jnp.float32)
    return pl.pallas_call(_tcab_body, out_shape=[vec, vec])(dr, dc)


def _zero_rows(buf, rows):
    z = jnp.zeros((16,), jnp.float32)

    @pl.loop(0, rows)
    def _(n):
        buf[n, pl.ds(0, 16)] = z


def _aggregate(vtab, colv, rowv, u, gbuf, gsems, ssem, nb):
    @pl.loop(0, NCH // nb)
    def _(og):
        gds = []
        for m in range(nb):
            j = og * nb + m
            gds.append(pltpu.async_copy(vtab.at[colv.at[j]], gbuf.at[m],
                                        gsems.at[m]))
        sds = []
        for m in range(nb):
            j = og * nb + m
            gds[m].wait()
            sds.append(pltpu.async_copy(gbuf.at[m], u.at[rowv.at[j]], ssem,
                                        add=True))
        for sd in sds:
            sd.wait()


def _combine(bufU, bufV, av, bv, first, prev, write_v):
    @pl.loop(0, NPN // 16)
    def _(g):
        a16 = av[pl.ds(16 * g, 16)]
        b16 = bv[pl.ds(16 * g, 16)]
        for i in range(16):
            n = 16 * g + i
            sl = pl.ds(0, 16)
            uu = bufU[n, sl]
            if first:
                t = a16[i] * uu
            else:
                t = 2.0 * (a16[i] * uu) - prev[n, sl]
            bufU[n, sl] = t
            if write_v:
                bufV[n, sl] = b16[i] * t


def _scheb_body(nt, nb, xh, er3, ec3, a_in, b_in,
                t1h, t2h, t3h, vh,
                rowv, colvs, gbuf, bufU, bufV, bufP, zbuf, av, bv,
                u, gsems, ssem):
    core = lax.axis_index("c")
    sub = lax.axis_index("s")
    base = sub * NPN

    pltpu.sync_copy(er3.at[sub], rowv)
    pltpu.sync_copy(a_in.at[pl.ds(base, NPN)], av)
    pltpu.sync_copy(b_in.at[pl.ds(base, NPN)], bv)

    _zero_rows(zbuf, NPN)
    pltpu.sync_copy(zbuf, u.at[pl.ds(base, NPN)])

    for t in range(nt):
        voff = (nt * core + t) * NPAD
        pltpu.sync_copy(ec3.at[sub], colvs.at[t])

        @pl.loop(0, NCH)
        def _(j, t=t, voff=voff):
            for m in range(ECH // 16):
                sl = pl.ds(16 * m, 16)
                colvs[t, j, sl] = colvs[t, j, sl] + voff

    for t in range(nt):
        tb = nt * core + t
        pltpu.sync_copy(xh.at[tb, pl.ds(base, NPN)], bufU)

        @pl.loop(0, NPN // 16)
        def _(g):
            b16 = bv[pl.ds(16 * g, 16)]
            for i in range(16):
                n = 16 * g + i
                bufV[n, pl.ds(0, 16)] = b16[i] * bufU[n, pl.ds(0, 16)]

        pltpu.sync_copy(bufV, vh.at[pl.ds(tb * NPAD + base, NPN)])

    plsc.subcore_barrier()

    for th, first, prev_src, write_v in (
            (t1h, True, None, True),
            (t2h, False, xh, True),
            (t3h, False, t1h, False)):
        for t in range(nt):
            tb = nt * core + t
            _aggregate(vh, colvs.at[t], rowv, u, gbuf, gsems, ssem, nb)
            plsc.subcore_barrier()
            pltpu.sync_copy(u.at[pl.ds(base, NPN)], bufU)
            pltpu.sync_copy(zbuf, u.at[pl.ds(base, NPN)])
            if prev_src is not None:
                pltpu.sync_copy(prev_src.at[tb, pl.ds(base, NPN)], bufP)
            _combine(bufU, bufV, av, bv, first, bufP, write_v)
            pltpu.sync_copy(bufU, th.at[tb, pl.ds(base, NPN)])
            if write_v:
                pltpu.sync_copy(bufV, vh.at[pl.ds(tb * NPAD + base, NPN)])
            plsc.subcore_barrier()


def _scheb(xh, er3, ec3, a_in, b_in, nt, nb):
    mesh = plsc.VectorSubcoreMesh(core_axis_name="c", subcore_axis_name="s")
    tabs = nt * NCORE
    half = jax.ShapeDtypeStruct((tabs, NPAD, CH), jnp.float32)
    out_type = [half, half, half,
                jax.ShapeDtypeStruct((tabs * NPAD, CH), jnp.float32)]
    scratch = [
        pltpu.VMEM((NCH, ECH), jnp.int32),
        pltpu.VMEM((nt, NCH, ECH), jnp.int32),
        pltpu.VMEM((nb, ECH, CH), jnp.float32),
        pltpu.VMEM((NPN, CH), jnp.float32),
        pltpu.VMEM((NPN, CH), jnp.float32),
        pltpu.VMEM((NPN, CH), jnp.float32),
        pltpu.VMEM((NPN, CH), jnp.float32),
        pltpu.VMEM((NPN,), jnp.float32),
        pltpu.VMEM((NPN,), jnp.float32),
        pltpu.VMEM_SHARED((NPAD, CH), jnp.float32),
        pltpu.SemaphoreType.DMA((nb,)),
        pltpu.SemaphoreType.DMA,
    ]
    fn = pl.kernel(functools.partial(_scheb_body, nt, nb), out_type=out_type,
                   mesh=mesh, scratch_types=scratch,
                   compiler_params=_SC_PARAMS)
    return fn(xh, er3, ec3, a_in, b_in)


_TC2G = 4
_NPB = NPAD // _TC2G


def _tc2_body(t0, t1, t2, t3, w1s, bias1, w2s, bias2, bn, vin, y2o, y1o, y0o):
    p3, p2, p1, p0 = [], [], [], []
    for bl in range(2):
        sl = slice(8 * bl, 8 * bl + 8)
        xcat = jnp.concatenate([t0[0][:, sl], t1[0][:, sl],
                                t2[0][:, sl], t3[0][:, sl]], axis=-1)
        x1 = _elu(xcat @ w1s[...] + bias1[...])
        y = x1 @ w2s[...]
        y0 = y[:, 0:8]
        y1 = y[:, 8:16]
        y2 = y[:, 16:24]
        y3 = y[:, 24:32]
        p3.append(4.0 * y3)
        p2.append(2.0 * y2)
        p1.append(y1 - 3.0 * y3)
        p0.append(y0 - y2 + bias2[...])
    bnb = bn[pl.ds(pl.program_id(1) * _NPB, _NPB)]
    vin[...] = bnb[:, None] * jnp.concatenate(p3, axis=-1)
    y2o[...] = jnp.concatenate(p2, axis=-1)
    y1o[...] = jnp.concatenate(p1, axis=-1)
    y0o[...] = jnp.concatenate(p0, axis=-1)


def _tc2(t0, t1, t2, t3, theta1, b1, theta2p, b2p, b_n):
    w1s = theta1.reshape(4 * F0, F1)
    w2s = theta2p.transpose(1, 0, 2).reshape(F1, 32)
    tin = lambda: pl.BlockSpec((1, _NPB, CH), lambda h, i: (h, i, 0))
    tout = pl.BlockSpec((_NPB, CH), lambda h, i: (h * _TC2G + i, 0))
    flat = jax.ShapeDtypeStruct((NCORE * NPAD, CH), jnp.float32)
    return pl.pallas_call(
        _tc2_body,
        grid=(NCORE, _TC2G),
        in_specs=[tin(), tin(), tin(), tin(),
                  pl.BlockSpec((4 * F0, F1), lambda h, i: (0, 0)),
                  pl.BlockSpec((F1,), lambda h, i: (0,)),
                  pl.BlockSpec((F1, 32), lambda h, i: (0, 0)),
                  pl.BlockSpec((8,), lambda h, i: (0,)),
                  pl.BlockSpec((NPAD,), lambda h, i: (0,))],
        out_specs=[tout, tout, tout, tout],
        out_shape=[flat, flat, flat, flat],
    )(t0, t1, t2, t3, w1s, b1, w2s, b2p, b_n)


def _tcfin_body(oh, out):
    for b in range(B):
        h, bl = divmod(b, 2)
        out[b] = oh[h][:, 8 * bl:8 * bl + FOUT]


def _tcfin(oh):
    nfb = N // 5
    return pl.pallas_call(
        _tcfin_body,
        grid=(5,),
        in_specs=[pl.BlockSpec((NCORE, nfb, CH), lambda i: (0, i, 0))],
        out_specs=pl.BlockSpec((B, nfb, FOUT), lambda i: (0, i, 0)),
        out_shape=jax.ShapeDtypeStruct((B, N, FOUT), jnp.float32),
    )(oh)


def _combine2(bufU, bufV, av, bv, addb, write_v):
    @pl.loop(0, NPN // 16)
    def _(g):
        a16 = av[pl.ds(16 * g, 16)]
        b16 = bv[pl.ds(16 * g, 16)]
        for i in range(16):
            n = 16 * g + i
            sl = pl.ds(0, 16)
            t = a16[i] * bufU[n, sl] + addb[n, sl]
            bufU[n, sl] = t
            if write_v:
                bufV[n, sl] = b16[i] * t


def _scheb2_body(nb, vin, er3, ec3, a_in, b_in, yt2, yt1, yt0,
                 out_h, vh,
                 rowv, colv, gbuf, bufU, bufV, bufP, zbuf, av, bv,
                 u, gsems, ssem):
    core = lax.axis_index("c")
    sub = lax.axis_index("s")
    base = sub * NPN
    voff = core * NPAD

    pltpu.sync_copy(er3.at[sub], rowv)
    pltpu.sync_copy(ec3.at[sub], colv)
    pltpu.sync_copy(a_in.at[pl.ds(base, NPN)], av)
    pltpu.sync_copy(b_in.at[pl.ds(base, NPN)], bv)

    @pl.loop(0, NCH)
    def _(j):
        for m in range(ECH // 16):
            sl = pl.ds(16 * m, 16)
            colv[j, sl] = colv[j, sl] + voff

    _zero_rows(zbuf, NPN)
    pltpu.sync_copy(zbuf, u.at[pl.ds(base, NPN)])
    plsc.subcore_barrier()

    for vtab, addtab, write_v in ((vin, yt2, True), (vh, yt1, True),
                                  (vh, yt0, False)):
        _aggregate(vtab, colv, rowv, u, gbuf, gsems, ssem, nb)
        plsc.subcore_barrier()
        pltpu.sync_copy(u.at[pl.ds(base, NPN)], bufU)
        pltpu.sync_copy(zbuf, u.at[pl.ds(base, NPN)])
        pltpu.sync_copy(addtab.at[pl.ds(voff + base, NPN)], bufP)
        _combine2(bufU, bufV, av, bv, bufP, write_v)
        if write_v:
            pltpu.sync_copy(bufV, vh.at[pl.ds(voff + base, NPN)])
        else:
            pltpu.sync_copy(bufU, out_h.at[core, pl.ds(base, NPN)])
        plsc.subcore_barrier()


def _scheb2(vinflat, er3, ec3, a_in, b_in, yt2, yt1, yt0, nb):
    mesh = plsc.VectorSubcoreMesh(core_axis_name="c", subcore_axis_name="s")
    out_type = [jax.ShapeDtypeStruct((NCORE, NPAD, CH), jnp.float32),
                jax.ShapeDtypeStruct((NCORE * NPAD, CH), jnp.float32)]
    scratch = [
        pltpu.VMEM((NCH, ECH), jnp.int32),
        pltpu.VMEM((NCH, ECH), jnp.int32),
        pltpu.VMEM((nb, ECH, CH), jnp.float32),
        pltpu.VMEM((NPN, CH), jnp.float32),
        pltpu.VMEM((NPN, CH), jnp.float32),
        pltpu.VMEM((NPN, CH), jnp.float32),
        pltpu.VMEM((NPN, CH), jnp.float32),
        pltpu.VMEM((NPN,), jnp.float32),
        pltpu.VMEM((NPN,), jnp.float32),
        pltpu.VMEM_SHARED((NPAD, CH), jnp.float32),
        pltpu.SemaphoreType.DMA((nb,)),
        pltpu.SemaphoreType.DMA,
    ]
    fn = pl.kernel(functools.partial(_scheb2_body, nb), out_type=out_type,
                   mesh=mesh, scratch_types=scratch,
                   compiler_params=_SC_PARAMS)
    return fn(vinflat, er3, ec3, a_in, b_in, yt2, yt1, yt0)


def kernel(mu, edge_row, edge_col, W_m0, b_m0, W_m1, b_m1, W_m2, b_m2,
           W_m3, b_m3, W_fc, b_fc, theta1, b1, theta2, b2):
    er3 = jnp.pad(edge_row, (0, EPAD - E), constant_values=N).reshape(
        NSUB, NCH, ECH)
    ec3 = jnp.pad(edge_col, (0, EPAD - E), constant_values=N).reshape(
        NSUB, NCH, ECH)
    est = jnp.stack([er3, ec3])

    theta2p = jnp.pad(theta2, ((0, 0), (0, 0), (0, 8 - FOUT)))
    b2p = jnp.pad(b2, (0, 8 - FOUT))

    dr, dc = _scdeg(est)
    a_n, b_n = _tcab(dr, dc)
    x0h = _tc1(mu, W_m0, b_m0, W_m1, b_m1, W_m2, b_m2, W_m3, b_m3,
               W_fc, b_fc)
    t1h, t2h, t3h, _v1 = _scheb(x0h, er3, ec3, a_n, b_n, 1, 16)
    vin, y2t, y1t, y0t = _tc2(x0h, t1h, t2h, t3h, theta1, b1, theta2p, b2p,
                              b_n)
    oh, _vh = _scheb2(vin, er3, ec3, a_n, b_n, y2t, y1t, y0t, 16)
    return _tcfin(oh)


# speedup vs baseline: 1.0311x; 1.0311x over previous
import functools

import jax
import jax.numpy as jnp
from jax import lax
from jax.experimental import pallas as pl
from jax.experimental.pallas import tpu as pltpu
from jax.experimental.pallas import tpu_sc as plsc

N = 10000
E = 160000
B = 4
HID = 64
LATENT = 64
F0 = 8
F1 = 16
FOUT = 3
CH = 16

NSUB = 16
NCORE = 2
NPN = 640
NPAD = NSUB * NPN
EPT = 10240
ECH = 128
NCH = EPT // ECH
EPAD = NSUB * EPT

_SC_PARAMS = pltpu.CompilerParams(use_tc_tiling_on_sc=False)


def _elu(x):
    return jnp.where(x > 0, x, jnp.exp(x) - 1.0)


def _scdeg_body(est, dr_out, dc_out, idxv, dz, ones, dg):
    core = lax.axis_index("c")
    sub = lax.axis_index("s")
    base = sub * NPN

    pltpu.sync_copy(est.at[core, sub], idxv)

    zf = jnp.zeros((16,), jnp.float32)
    one = jnp.ones((16,), jnp.float32)

    @pl.loop(0, NPN // 16)
    def _(i):
        dz[pl.ds(16 * i, 16)] = zf

    for q in range(ECH // 16):
        ones[pl.ds(16 * q, 16)] = one

    pltpu.sync_copy(dz, dg.at[pl.ds(base, NPN)])
    plsc.subcore_barrier()

    @pl.loop(0, NCH)
    def _(j):
        pltpu.sync_copy(ones, dg.at[idxv.at[j]], add=True)

    plsc.subcore_barrier()

    @pl.when(core == 0)
    def _():
        pltpu.sync_copy(dg.at[pl.ds(base, NPN)], dr_out.at[pl.ds(base, NPN)])

    @pl.when(core == 1)
    def _():
        pltpu.sync_copy(dg.at[pl.ds(base, NPN)], dc_out.at[pl.ds(base, NPN)])


def _scdeg(est):
    mesh = plsc.VectorSubcoreMesh(core_axis_name="c", subcore_axis_name="s")
    vec = jax.ShapeDtypeStruct((NPAD,), jnp.float32)
    scratch = [
        pltpu.VMEM((NCH, ECH), jnp.int32),
        pltpu.VMEM((NPN,), jnp.float32),
        pltpu.VMEM((ECH,), jnp.float32),
        pltpu.VMEM_SHARED((NPAD,), jnp.float32),
    ]
    fn = pl.kernel(_scdeg_body, out_type=[vec, vec], mesh=mesh,
                   scratch_types=scratch, compiler_params=_SC_PARAMS)
    return fn(est)


def _tc1_body(mu, w0, b0, w1, b1, w2, b2, w3, b3, wfc, bfc, out):
    h = _elu(mu[...] @ w0[...] + b0[...])
    h = _elu(h @ w1[...] + b1[...])
    h = _elu(h @ w2[...] + b2[...])
    z = h @ w3[...] + b3[...]
    x = z @ wfc[...] + bfc[...]
    xr = x.reshape(B, x.shape[1] // F0, F0)
    out[0] = jnp.concatenate([xr[0], xr[1]], axis=-1)
    out[1] = jnp.concatenate([xr[2], xr[3]], axis=-1)


def _tc1(mu, w0, b0, w1, b1, w2, b2, w3, b3, wfc, bfc):
    grid = 4
    npb = NPAD // grid
    cols = npb * F0
    full = lambda *shape: pl.BlockSpec(shape, lambda i: (0,) * len(shape))
    return pl.pallas_call(
        _tc1_body,
        grid=(grid,),
        in_specs=[
            full(B, 8), full(8, HID), full(HID,), full(HID, HID), full(HID,),
            full(HID, HID), full(HID,), full(HID, LATENT), full(LATENT,),
            pl.BlockSpec((LATENT, cols), lambda i: (0, i)),
            pl.BlockSpec((cols,), lambda i: (i,)),
        ],
        out_specs=pl.BlockSpec((NCORE, npb, CH), lambda i: (0, i, 0)),
        out_shape=jax.ShapeDtypeStruct((NCORE, NPAD, CH), jnp.float32),
    )(mu, w0, b0, w1, b1, w2, b2, w3, b3, wfc, bfc)


def _tcab_body(dr, dc, a_out, b_out):
    a_out[...] = -lax.rsqrt(jnp.maximum(dr[...], 1.0))
    b_out[...] = lax.rsqrt(jnp.maximum(dc[...], 1.0))


def _tcab(dr, dc):
    vec = jax.ShapeDtypeStruct((NPAD,), jnp.float32)
    return pl.pallas_call(_tcab_body, out_shape=[vec, vec])(dr, dc)


def _zero_rows(buf, rows):
    z = jnp.zeros((16,), jnp.float32)

    @pl.loop(0, rows)
    def _(n):
        buf[n, pl.ds(0, 16)] = z


def _aggregate(vtab, colv, rowv, u, gbuf, gsems, ssem, nb):
    @pl.loop(0, NCH // nb)
    def _(og):
        gds = []
        for m in range(nb):
            j = og * nb + m
            gds.append(pltpu.async_copy(vtab.at[colv.at[j]], gbuf.at[m],
                                        gsems.at[m]))
        sds = []
        for m in range(nb):
            j = og * nb + m
            gds[m].wait()
            sds.append(pltpu.async_copy(gbuf.at[m], u.at[rowv.at[j]], ssem,
                                        add=True))
        for sd in sds:
            sd.wait()


def _combine(bufU, bufV, av, bv, first, prev, write_v):
    @pl.loop(0, NPN // 16)
    def _(g):
        a16 = av[pl.ds(16 * g, 16)]
        b16 = bv[pl.ds(16 * g, 16)]
        for i in range(16):
            n = 16 * g + i
            sl = pl.ds(0, 16)
            uu = bufU[n, sl]
            if first:
                t = a16[i] * uu
            else:
                t = 2.0 * (a16[i] * uu) - prev[n, sl]
            bufU[n, sl] = t
            if write_v:
                bufV[n, sl] = b16[i] * t


def _scheb_body(nt, nb, xh, er3, ec3, a_in, b_in,
                t1h, t2h, t3h, vh,
                rowv, colvs, gbuf, bufU, bufV, bufP, zbuf, av, bv,
                u, gsems, ssem):
    core = lax.axis_index("c")
    sub = lax.axis_index("s")
    base = sub * NPN

    pltpu.sync_copy(er3.at[sub], rowv)
    pltpu.sync_copy(a_in.at[pl.ds(base, NPN)], av)
    pltpu.sync_copy(b_in.at[pl.ds(base, NPN)], bv)

    _zero_rows(zbuf, NPN)
    pltpu.sync_copy(zbuf, u.at[pl.ds(base, NPN)])

    for t in range(nt):
        voff = (nt * core + t) * NPAD
        pltpu.sync_copy(ec3.at[sub], colvs.at[t])

        @pl.loop(0, NCH)
        def _(j, t=t, voff=voff):
            for m in range(ECH // 16):
                sl = pl.ds(16 * m, 16)
                colvs[t, j, sl] = colvs[t, j, sl] + voff

    for t in range(nt):
        tb = nt * core + t
        pltpu.sync_copy(xh.at[tb, pl.ds(base, NPN)], bufU)

        @pl.loop(0, NPN // 16)
        def _(g):
            b16 = bv[pl.ds(16 * g, 16)]
            for i in range(16):
                n = 16 * g + i
                bufV[n, pl.ds(0, 16)] = b16[i] * bufU[n, pl.ds(0, 16)]

        pltpu.sync_copy(bufV, vh.at[pl.ds(tb * NPAD + base, NPN)])

    plsc.subcore_barrier()

    for th, first, prev_src, write_v in (
            (t1h, True, None, True),
            (t2h, False, xh, True),
            (t3h, False, t1h, False)):
        for t in range(nt):
            tb = nt * core + t
            _aggregate(vh, colvs.at[t], rowv, u, gbuf, gsems, ssem, nb)
            plsc.subcore_barrier()
            pltpu.sync_copy(u.at[pl.ds(base, NPN)], bufU)
            pltpu.sync_copy(zbuf, u.at[pl.ds(base, NPN)])
            if prev_src is not None:
                pltpu.sync_copy(prev_src.at[tb, pl.ds(base, NPN)], bufP)
            _combine(bufU, bufV, av, bv, first, bufP, write_v)
            pltpu.sync_copy(bufU, th.at[tb, pl.ds(base, NPN)])
            if write_v:
                pltpu.sync_copy(bufV, vh.at[pl.ds(tb * NPAD + base, NPN)])
            plsc.subcore_barrier()


def _scheb(xh, er3, ec3, a_in, b_in, nt, nb):
    mesh = plsc.VectorSubcoreMesh(core_axis_name="c", subcore_axis_name="s")
    tabs = nt * NCORE
    half = jax.ShapeDtypeStruct((tabs, NPAD, CH), jnp.float32)
    out_type = [half, half, half,
                jax.ShapeDtypeStruct((tabs * NPAD, CH), jnp.float32)]
    scratch = [
        pltpu.VMEM((NCH, ECH), jnp.int32),
        pltpu.VMEM((nt, NCH, ECH), jnp.int32),
        pltpu.VMEM((nb, ECH, CH), jnp.float32),
        pltpu.VMEM((NPN, CH), jnp.float32),
        pltpu.VMEM((NPN, CH), jnp.float32),
        pltpu.VMEM((NPN, CH), jnp.float32),
        pltpu.VMEM((NPN, CH), jnp.float32),
        pltpu.VMEM((NPN,), jnp.float32),
        pltpu.VMEM((NPN,), jnp.float32),
        pltpu.VMEM_SHARED((NPAD, CH), jnp.float32),
        pltpu.SemaphoreType.DMA((nb,)),
        pltpu.SemaphoreType.DMA,
    ]
    fn = pl.kernel(functools.partial(_scheb_body, nt, nb), out_type=out_type,
                   mesh=mesh, scratch_types=scratch,
                   compiler_params=_SC_PARAMS)
    return fn(xh, er3, ec3, a_in, b_in)


_TC2G = 4
_NPB = NPAD // _TC2G


def _tc2_body(t0, t1, t2, t3, th1, bias1, th2, bias2, bn, vin, y2o, y1o, y0o):
    p3, p2, p1, p0 = [], [], [], []
    for bl in range(2):
        sl = slice(8 * bl, 8 * bl + 8)
        x1 = _elu(t0[0][:, sl] @ th1[0] + t1[0][:, sl] @ th1[1]
                  + t2[0][:, sl] @ th1[2] + t3[0][:, sl] @ th1[3]
                  + bias1[...])
        y0 = x1 @ th2[0]
        y1 = x1 @ th2[1]
        y2 = x1 @ th2[2]
        y3 = x1 @ th2[3]
        p3.append(4.0 * y3)
        p2.append(2.0 * y2)
        p1.append(y1 - 3.0 * y3)
        p0.append(y0 - y2 + bias2[...])
    bnb = bn[pl.ds(pl.program_id(1) * _NPB, _NPB)]
    vin[...] = bnb[:, None] * jnp.concatenate(p3, axis=-1)
    y2o[...] = jnp.concatenate(p2, axis=-1)
    y1o[...] = jnp.concatenate(p1, axis=-1)
    y0o[...] = jnp.concatenate(p0, axis=-1)


def _tc2(t0, t1, t2, t3, theta1, b1, theta2p, b2p, b_n):
    tin = lambda: pl.BlockSpec((1, _NPB, CH), lambda h, i: (h, i, 0))
    tout = pl.BlockSpec((_NPB, CH), lambda h, i: (h * _TC2G + i, 0))
    flat = jax.ShapeDtypeStruct((NCORE * NPAD, CH), jnp.float32)
    return pl.pallas_call(
        _tc2_body,
        grid=(NCORE, _TC2G),
        in_specs=[tin(), tin(), tin(), tin(),
                  pl.BlockSpec((4, F0, F1), lambda h, i: (0, 0, 0)),
                  pl.BlockSpec((F1,), lambda h, i: (0,)),
                  pl.BlockSpec((4, F1, 8), lambda h, i: (0, 0, 0)),
                  pl.BlockSpec((8,), lambda h, i: (0,)),
                  pl.BlockSpec((NPAD,), lambda h, i: (0,))],
        out_specs=[tout, tout, tout, tout],
        out_shape=[flat, flat, flat, flat],
    )(t0, t1, t2, t3, theta1, b1, theta2p, b2p, b_n)


def _tcfin_body(oh, out):
    for b in range(B):
        h, bl = divmod(b, 2)
        out[b] = oh[h][:, 8 * bl:8 * bl + FOUT]


def _tcfin(oh):
    nfb = N // 5
    return pl.pallas_call(
        _tcfin_body,
        grid=(5,),
        in_specs=[pl.BlockSpec((NCORE, nfb, CH), lambda i: (0, i, 0))],
        out_specs=pl.BlockSpec((B, nfb, FOUT), lambda i: (0, i, 0)),
        out_shape=jax.ShapeDtypeStruct((B, N, FOUT), jnp.float32),
    )(oh)


def _combine2(bufU, bufV, av, bv, addb, write_v):
    @pl.loop(0, NPN // 16)
    def _(g):
        a16 = av[pl.ds(16 * g, 16)]
        b16 = bv[pl.ds(16 * g, 16)]
        for i in range(16):
            n = 16 * g + i
            sl = pl.ds(0, 16)
            t = a16[i] * bufU[n, sl] + addb[n, sl]
            bufU[n, sl] = t
            if write_v:
                bufV[n, sl] = b16[i] * t


def _scheb2_body(nb, vin, er3, ec3, a_in, b_in, yt2, yt1, yt0,
                 out_h, vh,
                 rowv, colv, gbuf, bufU, bufV, bufP, zbuf, av, bv,
                 u, gsems, ssem):
    core = lax.axis_index("c")
    sub = lax.axis_index("s")
    base = sub * NPN
    voff = core * NPAD

    pltpu.sync_copy(er3.at[sub], rowv)
    pltpu.sync_copy(ec3.at[sub], colv)
    pltpu.sync_copy(a_in.at[pl.ds(base, NPN)], av)
    pltpu.sync_copy(b_in.at[pl.ds(base, NPN)], bv)

    @pl.loop(0, NCH)
    def _(j):
        for m in range(ECH // 16):
            sl = pl.ds(16 * m, 16)
            colv[j, sl] = colv[j, sl] + voff

    _zero_rows(zbuf, NPN)
    pltpu.sync_copy(zbuf, u.at[pl.ds(base, NPN)])
    plsc.subcore_barrier()

    for vtab, addtab, write_v in ((vin, yt2, True), (vh, yt1, True),
                                  (vh, yt0, False)):
        _aggregate(vtab, colv, rowv, u, gbuf, gsems, ssem, nb)
        plsc.subcore_barrier()
        pltpu.sync_copy(u.at[pl.ds(base, NPN)], bufU)
        pltpu.sync_copy(zbuf, u.at[pl.ds(base, NPN)])
        pltpu.sync_copy(addtab.at[pl.ds(voff + base, NPN)], bufP)
        _combine2(bufU, bufV, av, bv, bufP, write_v)
        if write_v:
            pltpu.sync_copy(bufV, vh.at[pl.ds(voff + base, NPN)])
        else:
            pltpu.sync_copy(bufU, out_h.at[core, pl.ds(base, NPN)])
        plsc.subcore_barrier()


def _scheb2(vinflat, er3, ec3, a_in, b_in, yt2, yt1, yt0, nb):
    mesh = plsc.VectorSubcoreMesh(core_axis_name="c", subcore_axis_name="s")
    out_type = [jax.ShapeDtypeStruct((NCORE, NPAD, CH), jnp.float32),
                jax.ShapeDtypeStruct((NCORE * NPAD, CH), jnp.float32)]
    scratch = [
        pltpu.VMEM((NCH, ECH), jnp.int32),
        pltpu.VMEM((NCH, ECH), jnp.int32),
        pltpu.VMEM((nb, ECH, CH), jnp.float32),
        pltpu.VMEM((NPN, CH), jnp.float32),
        pltpu.VMEM((NPN, CH), jnp.float32),
        pltpu.VMEM((NPN, CH), jnp.float32),
        pltpu.VMEM((NPN, CH), jnp.float32),
        pltpu.VMEM((NPN,), jnp.float32),
        pltpu.VMEM((NPN,), jnp.float32),
        pltpu.VMEM_SHARED((NPAD, CH), jnp.float32),
        pltpu.SemaphoreType.DMA((nb,)),
        pltpu.SemaphoreType.DMA,
    ]
    fn = pl.kernel(functools.partial(_scheb2_body, nb), out_type=out_type,
                   mesh=mesh, scratch_types=scratch,
                   compiler_params=_SC_PARAMS)
    return fn(vinflat, er3, ec3, a_in, b_in, yt2, yt1, yt0)


def kernel(mu, edge_row, edge_col, W_m0, b_m0, W_m1, b_m1, W_m2, b_m2,
           W_m3, b_m3, W_fc, b_fc, theta1, b1, theta2, b2):
    er3 = jnp.pad(edge_row, (0, EPAD - E), constant_values=N).reshape(
        NSUB, NCH, ECH)
    ec3 = jnp.pad(edge_col, (0, EPAD - E), constant_values=N).reshape(
        NSUB, NCH, ECH)
    est = jnp.stack([er3, ec3])

    theta2p = jnp.pad(theta2, ((0, 0), (0, 0), (0, 8 - FOUT)))
    b2p = jnp.pad(b2, (0, 8 - FOUT))

    dr, dc = _scdeg(est)
    a_n, b_n = _tcab(dr, dc)
    x0h = _tc1(mu, W_m0, b_m0, W_m1, b_m1, W_m2, b_m2, W_m3, b_m3,
               W_fc, b_fc)
    t1h, t2h, t3h, _v1 = _scheb(x0h, er3, ec3, a_n, b_n, 1, 20)
    vin, y2t, y1t, y0t = _tc2(x0h, t1h, t2h, t3h, theta1, b1, theta2p, b2p,
                              b_n)
    oh, _vh = _scheb2(vin, er3, ec3, a_n, b_n, y2t, y1t, y0t, 20)
    return _tcfin(oh)


# speedup vs baseline: 1.0353x; 1.0041x over previous
import functools

import jax
import jax.numpy as jnp
from jax import lax
from jax.experimental import pallas as pl
from jax.experimental.pallas import tpu as pltpu
from jax.experimental.pallas import tpu_sc as plsc

N = 10000
E = 160000
B = 4
HID = 64
LATENT = 64
F0 = 8
F1 = 16
FOUT = 3
CH = 16

NSUB = 16
NCORE = 2
NPN = 640
NPAD = NSUB * NPN
EPT = 10240
ECH = 128
NCH = EPT // ECH
EPAD = NSUB * EPT

_SC_PARAMS = pltpu.CompilerParams(use_tc_tiling_on_sc=False)


def _elu(x):
    return jnp.where(x > 0, x, jnp.exp(x) - 1.0)


def _scdeg_body(est, dr_out, dc_out, idxv, dz, ones, dg):
    core = lax.axis_index("c")
    sub = lax.axis_index("s")
    base = sub * NPN

    pltpu.sync_copy(est.at[core, sub], idxv)

    zf = jnp.zeros((16,), jnp.float32)
    one = jnp.ones((16,), jnp.float32)

    @pl.loop(0, NPN // 16)
    def _(i):
        dz[pl.ds(16 * i, 16)] = zf

    for q in range(ECH // 16):
        ones[pl.ds(16 * q, 16)] = one

    pltpu.sync_copy(dz, dg.at[pl.ds(base, NPN)])
    plsc.subcore_barrier()

    @pl.loop(0, NCH)
    def _(j):
        pltpu.sync_copy(ones, dg.at[idxv.at[j]], add=True)

    plsc.subcore_barrier()

    @pl.when(core == 0)
    def _():
        pltpu.sync_copy(dg.at[pl.ds(base, NPN)], dr_out.at[pl.ds(base, NPN)])

    @pl.when(core == 1)
    def _():
        pltpu.sync_copy(dg.at[pl.ds(base, NPN)], dc_out.at[pl.ds(base, NPN)])


def _scdeg(est):
    mesh = plsc.VectorSubcoreMesh(core_axis_name="c", subcore_axis_name="s")
    vec = jax.ShapeDtypeStruct((NPAD,), jnp.float32)
    scratch = [
        pltpu.VMEM((NCH, ECH), jnp.int32),
        pltpu.VMEM((NPN,), jnp.float32),
        pltpu.VMEM((ECH,), jnp.float32),
        pltpu.VMEM_SHARED((NPAD,), jnp.float32),
    ]
    fn = pl.kernel(_scdeg_body, out_type=[vec, vec], mesh=mesh,
                   scratch_types=scratch, compiler_params=_SC_PARAMS)
    return fn(est)


def _tc1_body(mu, w0, b0, w1, b1, w2, b2, w3, b3, wfc, bfc, out):
    h = _elu(mu[...] @ w0[...] + b0[...])
    h = _elu(h @ w1[...] + b1[...])
    h = _elu(h @ w2[...] + b2[...])
    z = h @ w3[...] + b3[...]
    x = z @ wfc[...] + bfc[...]
    xr = x.reshape(B, x.shape[1] // F0, F0)
    out[0] = jnp.concatenate([xr[0], xr[1]], axis=-1)
    out[1] = jnp.concatenate([xr[2], xr[3]], axis=-1)


def _tc1(mu, w0, b0, w1, b1, w2, b2, w3, b3, wfc, bfc):
    grid = 4
    npb = NPAD // grid
    cols = npb * F0
    full = lambda *shape: pl.BlockSpec(shape, lambda i: (0,) * len(shape))
    return pl.pallas_call(
        _tc1_body,
        grid=(grid,),
        in_specs=[
            full(B, 8), full(8, HID), full(HID,), full(HID, HID), full(HID,),
            full(HID, HID), full(HID,), full(HID, LATENT), full(LATENT,),
            pl.BlockSpec((LATENT, cols), lambda i: (0, i)),
            pl.BlockSpec((cols,), lambda i: (i,)),
        ],
        out_specs=pl.BlockSpec((NCORE, npb, CH), lambda i: (0, i, 0)),
        out_shape=jax.ShapeDtypeStruct((NCORE, NPAD, CH), jnp.float32),
    )(mu, w0, b0, w1, b1, w2, b2, w3, b3, wfc, bfc)


def _tcab_body(dr, dc, a_out, b_out):
    a_out[...] = -lax.rsqrt(jnp.maximum(dr[...], 1.0))
    b_out[...] = lax.rsqrt(jnp.maximum(dc[...], 1.0))


def _tcab(dr, dc):
    vec = jax.ShapeDtypeStruct((NPAD,), jnp.float32)
    return pl.pallas_call(_tcab_body, out_shape=[vec, vec])(dr, dc)


def _zero_rows(buf, rows):
    z = jnp.zeros((16,), jnp.float32)

    @pl.loop(0, rows)
    def _(n):
        buf[n, pl.ds(0, 16)] = z


def _aggregate(vtab, colv, rowv, u, gbuf, gsems, ssem, nb):
    @pl.loop(0, NCH // nb)
    def _(og):
        gds = []
        for m in range(nb):
            j = og * nb + m
            gds.append(pltpu.async_copy(vtab.at[colv.at[j]], gbuf.at[m],
                                        gsems.at[m]))
        sds = []
        for m in range(nb):
            j = og * nb + m
            gds[m].wait()
            sds.append(pltpu.async_copy(gbuf.at[m], u.at[rowv.at[j]], ssem,
                                        add=True))
        for sd in sds:
            sd.wait()


def _combine(bufU, bufV, av, bv, first, prev, write_v):
    @pl.loop(0, NPN // 16)
    def _(g):
        a16 = av[pl.ds(16 * g, 16)]
        b16 = bv[pl.ds(16 * g, 16)]
        for i in range(16):
            n = 16 * g + i
            sl = pl.ds(0, 16)
            uu = bufU[n, sl]
            if first:
                t = a16[i] * uu
            else:
                t = 2.0 * (a16[i] * uu) - prev[n, sl]
            bufU[n, sl] = t
            if write_v:
                bufV[n, sl] = b16[i] * t


def _scheb_body(nt, nb, xh, er3, ec3, a_in, b_in,
                tout, vh,
                rowv, colvs, gbuf, bufU, bufV, bufP, zbuf, av, bv,
                u, gsems, ssem):
    core = lax.axis_index("c")
    sub = lax.axis_index("s")
    base = sub * NPN

    pltpu.sync_copy(er3.at[sub], rowv)
    pltpu.sync_copy(a_in.at[pl.ds(base, NPN)], av)
    pltpu.sync_copy(b_in.at[pl.ds(base, NPN)], bv)

    _zero_rows(zbuf, NPN)
    pltpu.sync_copy(zbuf, u.at[pl.ds(base, NPN)])

    for t in range(nt):
        voff = (nt * core + t) * NPAD
        pltpu.sync_copy(ec3.at[sub], colvs.at[t])

        @pl.loop(0, NCH)
        def _(j, t=t, voff=voff):
            for m in range(ECH // 16):
                sl = pl.ds(16 * m, 16)
                colvs[t, j, sl] = colvs[t, j, sl] + voff

    for t in range(nt):
        tb = nt * core + t
        pltpu.sync_copy(xh.at[tb, pl.ds(base, NPN)], bufU)

        @pl.loop(0, NPN // 16)
        def _(g):
            b16 = bv[pl.ds(16 * g, 16)]
            for i in range(16):
                n = 16 * g + i
                bufV[n, pl.ds(0, 16)] = b16[i] * bufU[n, pl.ds(0, 16)]

        pltpu.sync_copy(bufV, vh.at[pl.ds(tb * NPAD + base, NPN)])

    plsc.subcore_barrier()

    for k, first, write_v in ((0, True, True), (1, False, True),
                              (2, False, False)):
        for t in range(nt):
            tb = nt * core + t
            _aggregate(vh, colvs.at[t], rowv, u, gbuf, gsems, ssem, nb)
            plsc.subcore_barrier()
            pltpu.sync_copy(u.at[pl.ds(base, NPN)], bufU)
            pltpu.sync_copy(zbuf, u.at[pl.ds(base, NPN)])
            if k == 1:
                pltpu.sync_copy(xh.at[tb, pl.ds(base, NPN)], bufP)
            elif k == 2:
                pltpu.sync_copy(tout.at[0, tb, pl.ds(base, NPN)], bufP)
            _combine(bufU, bufV, av, bv, first, bufP, write_v)
            pltpu.sync_copy(bufU, tout.at[k, tb, pl.ds(base, NPN)])
            if write_v:
                pltpu.sync_copy(bufV, vh.at[pl.ds(tb * NPAD + base, NPN)])
            plsc.subcore_barrier()


def _scheb(xh, er3, ec3, a_in, b_in, nt, nb):
    mesh = plsc.VectorSubcoreMesh(core_axis_name="c", subcore_axis_name="s")
    tabs = nt * NCORE
    out_type = [jax.ShapeDtypeStruct((3, tabs, NPAD, CH), jnp.float32),
                jax.ShapeDtypeStruct((tabs * NPAD, CH), jnp.float32)]
    scratch = [
        pltpu.VMEM((NCH, ECH), jnp.int32),
        pltpu.VMEM((nt, NCH, ECH), jnp.int32),
        pltpu.VMEM((nb, ECH, CH), jnp.float32),
        pltpu.VMEM((NPN, CH), jnp.float32),
        pltpu.VMEM((NPN, CH), jnp.float32),
        pltpu.VMEM((NPN, CH), jnp.float32),
        pltpu.VMEM((NPN, CH), jnp.float32),
        pltpu.VMEM((NPN,), jnp.float32),
        pltpu.VMEM((NPN,), jnp.float32),
        pltpu.VMEM_SHARED((NPAD, CH), jnp.float32),
        pltpu.SemaphoreType.DMA((nb,)),
        pltpu.SemaphoreType.DMA,
    ]
    fn = pl.kernel(functools.partial(_scheb_body, nt, nb), out_type=out_type,
                   mesh=mesh, scratch_types=scratch,
                   compiler_params=_SC_PARAMS)
    return fn(xh, er3, ec3, a_in, b_in)


_TC2G = 4
_NPB = NPAD // _TC2G


def _tc2_body(t0, tt, th1, bias1, th2, bias2, bn, vin):
    p3, p2, p1, p0 = [], [], [], []
    for bl in range(2):
        sl = slice(8 * bl, 8 * bl + 8)
        x1 = _elu(t0[0][:, sl] @ th1[0] + tt[0][0][:, sl] @ th1[1]
                  + tt[1][0][:, sl] @ th1[2] + tt[2][0][:, sl] @ th1[3]
                  + bias1[...])
        y0 = x1 @ th2[0]
        y1 = x1 @ th2[1]
        y2 = x1 @ th2[2]
        y3 = x1 @ th2[3]
        p3.append(4.0 * y3)
        p2.append(2.0 * y2)
        p1.append(y1 - 3.0 * y3)
        p0.append(y0 - y2 + bias2[...])
    bnb = bn[pl.ds(pl.program_id(1) * _NPB, _NPB)]
    vin[0] = bnb[:, None] * jnp.concatenate(p3, axis=-1)
    vin[1] = jnp.concatenate(p2, axis=-1)
    vin[2] = jnp.concatenate(p1, axis=-1)
    vin[3] = jnp.concatenate(p0, axis=-1)


def _tc2(t0, tth, theta1, b1, theta2p, b2p, b_n):
    return pl.pallas_call(
        _tc2_body,
        grid=(NCORE, _TC2G),
        in_specs=[pl.BlockSpec((1, _NPB, CH), lambda h, i: (h, i, 0)),
                  pl.BlockSpec((3, 1, _NPB, CH), lambda h, i: (0, h, i, 0)),
                  pl.BlockSpec((4, F0, F1), lambda h, i: (0, 0, 0)),
                  pl.BlockSpec((F1,), lambda h, i: (0,)),
                  pl.BlockSpec((4, F1, 8), lambda h, i: (0, 0, 0)),
                  pl.BlockSpec((8,), lambda h, i: (0,)),
                  pl.BlockSpec((NPAD,), lambda h, i: (0,))],
        out_specs=pl.BlockSpec((4, _NPB, CH), lambda h, i: (0, h * _TC2G + i, 0)),
        out_shape=jax.ShapeDtypeStruct((4, NCORE * NPAD, CH), jnp.float32),
    )(t0, tth, theta1, b1, theta2p, b2p, b_n)


def _tcfin_body(oh, out):
    for b in range(B):
        h, bl = divmod(b, 2)
        out[b] = oh[h][:, 8 * bl:8 * bl + FOUT]


def _tcfin(oh):
    nfb = N // 5
    return pl.pallas_call(
        _tcfin_body,
        grid=(5,),
        in_specs=[pl.BlockSpec((NCORE, nfb, CH), lambda i: (0, i, 0))],
        out_specs=pl.BlockSpec((B, nfb, FOUT), lambda i: (0, i, 0)),
        out_shape=jax.ShapeDtypeStruct((B, N, FOUT), jnp.float32),
    )(oh)


def _combine2(bufU, bufV, av, bv, addb, write_v):
    @pl.loop(0, NPN // 16)
    def _(g):
        a16 = av[pl.ds(16 * g, 16)]
        b16 = bv[pl.ds(16 * g, 16)]
        for i in range(16):
            n = 16 * g + i
            sl = pl.ds(0, 16)
            t = a16[i] * bufU[n, sl] + addb[n, sl]
            bufU[n, sl] = t
            if write_v:
                bufV[n, sl] = b16[i] * t


def _scheb2_body(nb, yt, er3, ec3, a_in, b_in,
                 out_h, vh,
                 rowv, colv, gbuf, bufU, bufV, bufP, zbuf, av, bv,
                 u, gsems, ssem):
    core = lax.axis_index("c")
    sub = lax.axis_index("s")
    base = sub * NPN
    voff = core * NPAD

    pltpu.sync_copy(er3.at[sub], rowv)
    pltpu.sync_copy(ec3.at[sub], colv)
    pltpu.sync_copy(a_in.at[pl.ds(base, NPN)], av)
    pltpu.sync_copy(b_in.at[pl.ds(base, NPN)], bv)

    @pl.loop(0, NCH)
    def _(j):
        for m in range(ECH // 16):
            sl = pl.ds(16 * m, 16)
            colv[j, sl] = colv[j, sl] + voff

    _zero_rows(zbuf, NPN)
    pltpu.sync_copy(zbuf, u.at[pl.ds(base, NPN)])
    plsc.subcore_barrier()

    for vtab, seg, write_v in ((yt.at[0], 1, True), (vh, 2, True),
                               (vh, 3, False)):
        _aggregate(vtab, colv, rowv, u, gbuf, gsems, ssem, nb)
        plsc.subcore_barrier()
        pltpu.sync_copy(u.at[pl.ds(base, NPN)], bufU)
        pltpu.sync_copy(zbuf, u.at[pl.ds(base, NPN)])
        pltpu.sync_copy(yt.at[seg, pl.ds(voff + base, NPN)], bufP)
        _combine2(bufU, bufV, av, bv, bufP, write_v)
        if write_v:
            pltpu.sync_copy(bufV, vh.at[pl.ds(voff + base, NPN)])
        else:
            pltpu.sync_copy(bufU, out_h.at[core, pl.ds(base, NPN)])
        plsc.subcore_barrier()


def _scheb2(yt, er3, ec3, a_in, b_in, nb):
    mesh = plsc.VectorSubcoreMesh(core_axis_name="c", subcore_axis_name="s")
    out_type = [jax.ShapeDtypeStruct((NCORE, NPAD, CH), jnp.float32),
                jax.ShapeDtypeStruct((NCORE * NPAD, CH), jnp.float32)]
    scratch = [
        pltpu.VMEM((NCH, ECH), jnp.int32),
        pltpu.VMEM((NCH, ECH), jnp.int32),
        pltpu.VMEM((nb, ECH, CH), jnp.float32),
        pltpu.VMEM((NPN, CH), jnp.float32),
        pltpu.VMEM((NPN, CH), jnp.float32),
        pltpu.VMEM((NPN, CH), jnp.float32),
        pltpu.VMEM((NPN, CH), jnp.float32),
        pltpu.VMEM((NPN,), jnp.float32),
        pltpu.VMEM((NPN,), jnp.float32),
        pltpu.VMEM_SHARED((NPAD, CH), jnp.float32),
        pltpu.SemaphoreType.DMA((nb,)),
        pltpu.SemaphoreType.DMA,
    ]
    fn = pl.kernel(functools.partial(_scheb2_body, nb), out_type=out_type,
                   mesh=mesh, scratch_types=scratch,
                   compiler_params=_SC_PARAMS)
    return fn(yt, er3, ec3, a_in, b_in)


def kernel(mu, edge_row, edge_col, W_m0, b_m0, W_m1, b_m1, W_m2, b_m2,
           W_m3, b_m3, W_fc, b_fc, theta1, b1, theta2, b2):
    er3 = jnp.pad(edge_row, (0, EPAD - E), constant_values=N).reshape(
        NSUB, NCH, ECH)
    ec3 = jnp.pad(edge_col, (0, EPAD - E), constant_values=N).reshape(
        NSUB, NCH, ECH)
    est = jnp.stack([er3, ec3])

    theta2p = jnp.pad(theta2, ((0, 0), (0, 0), (0, 8 - FOUT)))
    b2p = jnp.pad(b2, (0, 8 - FOUT))

    dr, dc = _scdeg(est)
    a_n, b_n = _tcab(dr, dc)
    x0h = _tc1(mu, W_m0, b_m0, W_m1, b_m1, W_m2, b_m2, W_m3, b_m3,
               W_fc, b_fc)
    tth, _v1 = _scheb(x0h, er3, ec3, a_n, b_n, 1, 20)
    yt = _tc2(x0h, tth, theta1, b1, theta2p, b2p, b_n)
    oh, _vh = _scheb2(yt, er3, ec3, a_n, b_n, 20)
    return _tcfin(oh)
